# Initial kernel scaffold; baseline (speedup 1.0000x reference)
import jax
import jax.numpy as jnp
from jax.experimental import pallas as pl


def kernel(x, edge_index, edge_weight, W1, b1, g1, be1, W2, b2, g2, be2, W_ih, W_hh, b_ih, b_hh, Wo, bo):
    raise NotImplementedError("write your pallas kernel here")


# speedup vs baseline: 10.5916x; 10.5916x over previous
import functools

import jax
import jax.numpy as jnp
from jax import lax
from jax.experimental import pallas as pl
from jax.experimental.pallas import tpu as pltpu
from jax.experimental.pallas import tpu_sc as plsc

_NCORES = 2
_NTILES = 16
_CHUNK = 128


def _make_deg_dinv(NP, EP):
    epw = EP // _NTILES
    nchunks = epw // _CHUNK
    npt = NP // _NTILES
    mesh = plsc.VectorSubcoreMesh(core_axis_name="c", subcore_axis_name="s")

    @functools.partial(
        pl.kernel,
        out_type=jax.ShapeDtypeStruct((NP,), jnp.float32),
        mesh=mesh,
        compiler_params=pltpu.CompilerParams(needs_layout_passes=False),
        scratch_types=[
            pltpu.VMEM((_CHUNK,), jnp.int32),
            pltpu.VMEM((_CHUNK,), jnp.float32),
            pltpu.VMEM((npt,), jnp.float32),
            pltpu.VMEM_SHARED((NP,), jnp.float32),
        ],
    )
    def deg_kernel(dst_hbm, w_hbm, dinv_hbm, dstv, wv, dbuf, acc):
        cid = lax.axis_index("c")
        sid = lax.axis_index("s")

        @pl.when(cid == 0)
        def _():
            zv = jnp.zeros((16,), jnp.float32)

            def zb(i, _):
                dbuf[pl.ds(i * 16, 16)] = zv
                return 0

            lax.fori_loop(0, npt // 16, zb, 0)
            pltpu.sync_copy(dbuf, acc.at[pl.ds(sid * npt, npt)])
            plsc.subcore_barrier()

            def chunk(j, _):
                base = sid * epw + j * _CHUNK
                pltpu.sync_copy(dst_hbm.at[pl.ds(base, _CHUNK)], dstv)
                pltpu.sync_copy(w_hbm.at[pl.ds(base, _CHUNK)], wv)
                pltpu.sync_copy(wv, acc.at[dstv], add=True)
                return 0

            lax.fori_loop(0, nchunks, chunk, 0)
            plsc.subcore_barrier()
            pltpu.sync_copy(acc.at[pl.ds(sid * npt, npt)],
                            dinv_hbm.at[pl.ds(sid * npt, npt)])

    return deg_kernel


def _dinv_body(deg_ref, o_ref):
    o_ref[...] = lax.rsqrt(deg_ref[...] + 1.0)


def _make_agg(NP, EP, F):
    epw = EP // (_NCORES * _NTILES)
    nchunks = epw // _CHUNK
    npt = NP // _NTILES
    mesh = plsc.VectorSubcoreMesh(core_axis_name="c", subcore_axis_name="s")

    @functools.partial(
        pl.kernel,
        out_type=jax.ShapeDtypeStruct((_NCORES, NP, F), jnp.float32),
        mesh=mesh,
        compiler_params=pltpu.CompilerParams(needs_layout_passes=False),
        scratch_types=[
            pltpu.VMEM((_CHUNK,), jnp.int32),
            pltpu.VMEM((_CHUNK,), jnp.int32),
            pltpu.VMEM((_CHUNK,), jnp.float32),
            pltpu.VMEM((_CHUNK, F), jnp.float32),
            pltpu.VMEM_SHARED((NP, F), jnp.float32),
            pltpu.SemaphoreType.DMA,
        ],
    )
    def agg_kernel(h_hbm, src_hbm, dst_hbm, w_hbm, out_hbm,
                   srcv, dstv, wv, rows, acc, sem):
        cid = lax.axis_index("c")
        sid = lax.axis_index("s")
        zv = jnp.zeros((16,), jnp.float32)

        def zb(r, _):
            for l in range(F // 16):
                rows[r, pl.ds(l * 16, 16)] = zv
            return 0

        lax.fori_loop(0, _CHUNK, zb, 0)
        for i in range(npt // _CHUNK):
            pltpu.sync_copy(rows, acc.at[pl.ds(sid * npt + i * _CHUNK, _CHUNK)])
        plsc.subcore_barrier()

        base0 = (cid * _NTILES + sid) * epw

        def chunk(j, _):
            base = base0 + j * _CHUNK
            pltpu.sync_copy(src_hbm.at[pl.ds(base, _CHUNK)], srcv)
            pltpu.sync_copy(dst_hbm.at[pl.ds(base, _CHUNK)], dstv)
            pltpu.sync_copy(w_hbm.at[pl.ds(base, _CHUNK)], wv)
            pltpu.async_copy(h_hbm.at[srcv], rows, sem).wait()

            def scale(r, _):
                wr = plsc.load_gather(wv, [jnp.full((16,), r, jnp.int32)])
                for l in range(F // 16):
                    rows[r, pl.ds(l * 16, 16)] = rows[r, pl.ds(l * 16, 16)] * wr
                return 0

            lax.fori_loop(0, _CHUNK, scale, 0)
            pltpu.sync_copy(rows, acc.at[dstv], add=True)
            return 0

        lax.fori_loop(0, nchunks, chunk, 0)
        plsc.subcore_barrier()
        pltpu.sync_copy(acc.at[pl.ds(sid * npt, npt)],
                        out_hbm.at[cid, pl.ds(sid * npt, npt)])

    return agg_kernel


def _mm_scale_body(x_ref, w_ref, dinv_ref, o_ref):
    h = lax.dot_general(x_ref[...], w_ref[...], (((1,), (1,)), ((), ())),
                        preferred_element_type=jnp.float32)
    o_ref[...] = h * dinv_ref[...]


def _post_body(nvalid, R, h_ref, p_ref, dinv_ref, b_ref, y_ref, st_ref, acc):
    pid = pl.program_id(0)

    @pl.when(pid == 0)
    def _():
        acc[...] = jnp.zeros_like(acc)

    y = (p_ref[0] + p_ref[1] + h_ref[...]) * dinv_ref[...] + b_ref[...]
    y_ref[...] = y
    rowid = lax.broadcasted_iota(jnp.int32, (R, 1), 0) + pid * R
    m = (rowid < nvalid).astype(jnp.float32)
    ym = y * m
    acc[0:1, :] += jnp.sum(ym, axis=0, keepdims=True)
    acc[1:2, :] += jnp.sum(ym * y, axis=0, keepdims=True)

    @pl.when(pid == pl.num_programs(0) - 1)
    def _():
        st_ref[...] = jnp.zeros_like(st_ref)
        st_ref[0:2, :] = acc[...]


def _norm_mm_body(nvalid, scale_out, y_ref, st_ref, g_ref, be_ref, w_ref,
                  dinv_ref, b1_ref, b2_ref, o_ref):
    inv_n = 1.0 / nvalid
    mu = st_ref[0:1, :] * inv_n
    var = st_ref[1:2, :] * inv_n - mu * mu
    istd = lax.rsqrt(var + 1e-5)
    xn = (y_ref[...] - mu) * istd * g_ref[...] + be_ref[...]
    r = jnp.maximum(xn, 0.0)
    h = lax.dot_general(r, w_ref[...], (((1,), (1,)), ((), ())),
                        preferred_element_type=jnp.float32)
    if scale_out:
        o_ref[...] = h * dinv_ref[...]
    else:
        o_ref[...] = h + b1_ref[...] + b2_ref[...]


def _lstm_body(C, H, z_ref, whh_ref, wo_ref, bo_ref, o_ref, h_s, c_s, hs_buf):
    pid = pl.program_id(0)

    @pl.when(pid == 0)
    def _():
        h_s[...] = jnp.zeros_like(h_s)
        c_s[...] = jnp.zeros_like(c_s)

    whh = whh_ref[...]

    def step(t, carry):
        h, c = carry
        zt = z_ref[pl.ds(t, 1), :]
        z = zt + lax.dot_general(h, whh, (((1,), (1,)), ((), ())),
                                 preferred_element_type=jnp.float32)
        gi = jax.nn.sigmoid(z[:, 0:H])
        gf = jax.nn.sigmoid(z[:, H:2 * H])
        gg = jnp.tanh(z[:, 2 * H:3 * H])
        go = jax.nn.sigmoid(z[:, 3 * H:4 * H])
        c2 = gf * c + gi * gg
        h2 = go * jnp.tanh(c2)
        hs_buf[pl.ds(t, 1), :] = h2
        return (h2, c2)

    hf, cf = lax.fori_loop(0, C, step, (h_s[...], c_s[...]))
    h_s[...] = hf
    c_s[...] = cf
    out = lax.dot_general(hs_buf[...], wo_ref[...], (((1,), (1,)), ((), ())),
                          preferred_element_type=jnp.float32)
    o_ref[...] = out + bo_ref[...]


def kernel(x, edge_index, edge_weight, W1, b1, g1, be1, W2, b2, g2, be2,
           W_ih, W_hh, b_ih, b_hh, Wo, bo):
    N, D = x.shape
    H = W1.shape[0]
    E = edge_index.shape[1]
    P = Wo.shape[0]
    R = 1024
    NP = -(-N // R) * R
    EPU = _NCORES * _NTILES * _CHUNK
    EP = -(-E // EPU) * EPU

    src = edge_index[0]
    dst = edge_index[1]
    pe = EP - E
    if pe:
        src = jnp.concatenate([src, jnp.zeros((pe,), src.dtype)])
        dst = jnp.concatenate([dst, jnp.zeros((pe,), dst.dtype)])
        ew = jnp.concatenate([edge_weight, jnp.zeros((pe,), edge_weight.dtype)])
    else:
        ew = edge_weight
    xp = jnp.pad(x, ((0, NP - N), (0, 0)))

    deg = _make_deg_dinv(NP, EP)(dst, ew)
    dinv = pl.pallas_call(
        _dinv_body,
        out_shape=jax.ShapeDtypeStruct((NP // 128, 128), jnp.float32),
    )(deg.reshape(NP // 128, 128))
    dinv_col = dinv.reshape(NP, 1)

    G = NP // R
    row_spec = pl.BlockSpec((R, H), lambda i: (i, 0))
    col_spec = pl.BlockSpec((R, 1), lambda i: (i, 0))
    w_spec = lambda shp: pl.BlockSpec(shp, lambda i: (0,) * len(shp))

    h1s = pl.pallas_call(
        _mm_scale_body,
        grid=(G,),
        in_specs=[pl.BlockSpec((R, D), lambda i: (i, 0)), w_spec((H, D)), col_spec],
        out_specs=row_spec,
        out_shape=jax.ShapeDtypeStruct((NP, H), jnp.float32),
    )(xp, W1, dinv_col)

    agg = _make_agg(NP, EP, H)

    def post(hs, p, b):
        return pl.pallas_call(
            functools.partial(_post_body, N, R),
            grid=(G,),
            in_specs=[row_spec,
                      pl.BlockSpec((2, R, H), lambda i: (0, i, 0)),
                      col_spec, w_spec((1, H))],
            out_specs=[row_spec, pl.BlockSpec((8, H), lambda i: (0, 0))],
            out_shape=[jax.ShapeDtypeStruct((NP, H), jnp.float32),
                       jax.ShapeDtypeStruct((8, H), jnp.float32)],
            scratch_shapes=[pltpu.VMEM((2, H), jnp.float32)],
        )(hs, p, dinv_col, b.reshape(1, H))

    def norm_mm(y, st, g, be, w, b1b, b2b, scale_out, KOUT):
        return pl.pallas_call(
            functools.partial(_norm_mm_body, N, scale_out),
            grid=(G,),
            in_specs=[row_spec, w_spec((8, H)), w_spec((1, H)), w_spec((1, H)),
                      w_spec((KOUT, H)), col_spec, w_spec((1, KOUT)),
                      w_spec((1, KOUT))],
            out_specs=pl.BlockSpec((R, KOUT), lambda i: (i, 0)),
            out_shape=jax.ShapeDtypeStruct((NP, KOUT), jnp.float32),
        )(y, st, g.reshape(1, H), be.reshape(1, H), w, dinv_col,
          b1b.reshape(1, KOUT), b2b.reshape(1, KOUT))

    p1 = agg(h1s, src, dst, ew)
    y1, st1 = post(h1s, p1, b1)
    zero_h = jnp.zeros((H,), jnp.float32)
    h2s = norm_mm(y1, st1, g1, be1, W2, zero_h, zero_h, True, H)
    p2 = agg(h2s, src, dst, ew)
    y2, st2 = post(h2s, p2, b2)
    zpre = norm_mm(y2, st2, g2, be2, W_ih, b_ih, b_hh, False, 4 * H)

    C = 1000 if N % 1000 == 0 else N
    GL = N // C
    out = pl.pallas_call(
        functools.partial(_lstm_body, C, H),
        grid=(GL,),
        in_specs=[pl.BlockSpec((C, 4 * H), lambda i: (i, 0)),
                  w_spec((4 * H, H)), w_spec((P, H)), w_spec((1, P))],
        out_specs=pl.BlockSpec((C, P), lambda i: (i, 0)),
        out_shape=jax.ShapeDtypeStruct((N, P), jnp.float32),
        scratch_shapes=[pltpu.VMEM((1, H), jnp.float32),
                        pltpu.VMEM((1, H), jnp.float32),
                        pltpu.VMEM((C, H), jnp.float32)],
    )(zpre, W_hh, Wo, bo.reshape(1, P))
    return out


# speedup vs baseline: 13.6979x; 1.2933x over previous
import functools

import jax
import jax.numpy as jnp
from jax import lax
from jax.experimental import pallas as pl
from jax.experimental.pallas import tpu as pltpu
from jax.experimental.pallas import tpu_sc as plsc

_NCORES = 2
_NTILES = 16
_CHUNK = 120


def _make_deg_dinv(NP, EP):
    nch = EP // (_NCORES * _NTILES * _CHUNK)
    npt = NP // _NTILES
    mesh = plsc.VectorSubcoreMesh(core_axis_name="c", subcore_axis_name="s")

    @functools.partial(
        pl.kernel,
        out_type=jax.ShapeDtypeStruct((_NCORES * NP,), jnp.float32),
        mesh=mesh,
        compiler_params=pltpu.CompilerParams(needs_layout_passes=False),
        scratch_types=(
            [pltpu.VMEM((_CHUNK,), jnp.int32)] * 6
            + [pltpu.VMEM((_CHUNK,), jnp.float32)] * 6
            + [pltpu.VMEM((npt,), jnp.float32)]
            + [pltpu.VMEM_SHARED((NP,), jnp.float32)]
            + [pltpu.SemaphoreType.DMA] * 12
        ),
    )
    def deg_kernel(dst_hbm, w_hbm, deg_hbm, *refs):
        dstb = refs[0:6]
        wb = refs[6:12]
        dbuf = refs[12]
        acc = refs[13]
        esems = refs[14:20]
        ssems = refs[20:26]
        cid = lax.axis_index("c")
        sid = lax.axis_index("s")
        zv = jnp.zeros((16,), jnp.float32)

        def zb(i, _):
            dbuf[pl.ds(i * 16, 16)] = zv
            return 0

        lax.fori_loop(0, npt // 16, zb, 0)
        pltpu.sync_copy(dbuf, acc.at[pl.ds(sid * npt, npt)])
        plsc.subcore_barrier()

        tb = (cid * _NTILES + sid) * nch

        def ebase(j):
            return pl.multiple_of((tb + j) * _CHUNK, 8)

        def issue_e(j, e):
            pltpu.async_copy(dst_hbm.at[pl.ds(ebase(j), _CHUNK)], dstb[e],
                             esems[e])
            pltpu.async_copy(w_hbm.at[pl.ds(ebase(j), _CHUNK)], wb[e],
                             esems[e])

        def wait_e(j, e):
            pltpu.make_async_copy(dst_hbm.at[pl.ds(ebase(j), _CHUNK)],
                                  dstb[e], esems[e]).wait()
            pltpu.make_async_copy(w_hbm.at[pl.ds(ebase(j), _CHUNK)],
                                  wb[e], esems[e]).wait()

        def issue_s(j, e):
            pltpu.async_copy(wb[e], acc.at[dstb[e]], ssems[e], add=True)

        def wait_s(j, e):
            pltpu.make_async_copy(wb[e], acc.at[dstb[e]], ssems[e]).wait()

        for k in range(5):
            issue_e(k, k)

        def outer(jo, _):
            for b in range(6):
                j = jo * 6 + b
                wait_e(j, b)
                issue_s(j, b)

                @pl.when(j >= 1)
                def _():
                    wait_s(j - 1, (b + 5) % 6)

                @pl.when(j + 5 < nch)
                def _():
                    issue_e(j + 5, (b + 5) % 6)
            return 0

        lax.fori_loop(0, nch // 6, outer, 0)
        wait_s(nch - 1, (nch - 1) % 6)
        plsc.subcore_barrier()
        pltpu.sync_copy(acc.at[pl.ds(sid * npt, npt)],
                        deg_hbm.at[pl.ds(cid * NP + sid * npt, npt)])

    return deg_kernel


def _dinv_body(deg_ref, o_ref):
    o_ref[...] = lax.rsqrt(deg_ref[0:1, :] + deg_ref[1:2, :] + 1.0)


def _make_agg(NP, EP, F):
    nch = EP // (_NCORES * _NTILES * _CHUNK)
    npt = NP // _NTILES
    mesh = plsc.VectorSubcoreMesh(core_axis_name="c", subcore_axis_name="s")

    @functools.partial(
        pl.kernel,
        out_type=jax.ShapeDtypeStruct((_NCORES, NP, F), jnp.float32),
        mesh=mesh,
        compiler_params=pltpu.CompilerParams(needs_layout_passes=False),
        scratch_types=(
            [pltpu.VMEM((_CHUNK, F), jnp.float32)] * 3
            + [pltpu.VMEM((_CHUNK,), jnp.int32)] * 6
            + [pltpu.VMEM((_CHUNK,), jnp.int32)] * 6
            + [pltpu.VMEM((_CHUNK,), jnp.float32)] * 6
            + [pltpu.VMEM_SHARED((NP, F), jnp.float32)]
            + [pltpu.SemaphoreType.DMA] * 12
        ),
    )
    def agg_kernel(h_hbm, src_hbm, dst_hbm, w_hbm, out_hbm, *refs):
        bufs = refs[0:3]
        srcb = refs[3:9]
        dstb = refs[9:15]
        wb = refs[15:21]
        acc = refs[21]
        gsems = refs[22:25]
        ssems = refs[25:28]
        esems = refs[28:34]
        cid = lax.axis_index("c")
        sid = lax.axis_index("s")
        zv = jnp.zeros((16,), jnp.float32)

        def zb(r, _):
            for l in range(F // 16):
                bufs[0][r, pl.ds(l * 16, 16)] = zv
            return 0

        lax.fori_loop(0, _CHUNK, zb, 0)
        nfull, rem = npt // _CHUNK, npt % _CHUNK
        for i in range(nfull):
            pltpu.sync_copy(bufs[0],
                            acc.at[pl.ds(sid * npt + i * _CHUNK, _CHUNK)])
        if rem:
            pltpu.sync_copy(bufs[0].at[pl.ds(0, rem)],
                            acc.at[pl.ds(sid * npt + nfull * _CHUNK, rem)])
        plsc.subcore_barrier()

        tb = (cid * _NTILES + sid) * nch

        def ebase(j):
            return pl.multiple_of((tb + j) * _CHUNK, 8)

        def issue_e(j, e):
            pltpu.async_copy(src_hbm.at[pl.ds(ebase(j), _CHUNK)], srcb[e],
                             esems[e])
            pltpu.async_copy(dst_hbm.at[pl.ds(ebase(j), _CHUNK)], dstb[e],
                             esems[e])
            pltpu.async_copy(w_hbm.at[pl.ds(ebase(j), _CHUNK)], wb[e],
                             esems[e])

        def wait_e(j, e):
            pltpu.make_async_copy(src_hbm.at[pl.ds(ebase(j), _CHUNK)],
                                  srcb[e], esems[e]).wait()
            pltpu.make_async_copy(dst_hbm.at[pl.ds(ebase(j), _CHUNK)],
                                  dstb[e], esems[e]).wait()
            pltpu.make_async_copy(w_hbm.at[pl.ds(ebase(j), _CHUNK)],
                                  wb[e], esems[e]).wait()

        def issue_g(j, b, e):
            pltpu.async_copy(h_hbm.at[srcb[e]], bufs[b], gsems[b])

        def wait_g(j, b, e):
            pltpu.make_async_copy(h_hbm.at[srcb[e]], bufs[b], gsems[b]).wait()

        def issue_s(j, b, e):
            pltpu.async_copy(bufs[b], acc.at[dstb[e]], ssems[b], add=True)

        def wait_s(j, b, e):
            pltpu.make_async_copy(bufs[b], acc.at[dstb[e]], ssems[b]).wait()

        for k in range(5):
            issue_e(k, k)
        for k in range(2):
            wait_e(k, k)
            issue_g(k, k, k)

        def outer(jo, _):
            for b in range(6):
                j = jo * 6 + b
                bb = b % 3
                eb = b % 6
                wait_g(j, bb, eb)

                def scale(r4, _):
                    for u in range(4):
                        r = r4 * 4 + u
                        wr = plsc.load_gather(
                            wb[eb], [jnp.full((16,), r, jnp.int32)])
                        for l in range(F // 16):
                            bufs[bb][r, pl.ds(l * 16, 16)] = (
                                bufs[bb][r, pl.ds(l * 16, 16)] * wr)
                    return 0

                lax.fori_loop(0, _CHUNK // 4, scale, 0)
                issue_s(j, bb, eb)

                @pl.when(j >= 1)
                def _():
                    wait_s(j - 1, (bb + 2) % 3, (eb + 5) % 6)

                @pl.when(j + 5 < nch)
                def _():
                    issue_e(j + 5, (eb + 5) % 6)

                @pl.when(j + 2 < nch)
                def _():
                    wait_e(j + 2, (eb + 2) % 6)
                    issue_g(j + 2, (bb + 2) % 3, (eb + 2) % 6)
            return 0

        lax.fori_loop(0, nch // 6, outer, 0)
        wait_s(nch - 1, (nch - 1) % 3, (nch - 1) % 6)
        plsc.subcore_barrier()
        pltpu.sync_copy(acc.at[pl.ds(sid * npt, npt)],
                        out_hbm.at[cid, pl.ds(sid * npt, npt)])

    return agg_kernel


def _mm_scale_body(x_ref, w_ref, dinv_ref, o_ref):
    h = lax.dot_general(x_ref[...], w_ref[...], (((1,), (1,)), ((), ())),
                        preferred_element_type=jnp.float32)
    o_ref[...] = h * dinv_ref[...]


def _post_body(nvalid, R, h_ref, p_ref, dinv_ref, b_ref, y_ref, st_ref, acc):
    pid = pl.program_id(0)

    @pl.when(pid == 0)
    def _():
        acc[...] = jnp.zeros_like(acc)

    y = (p_ref[0] + p_ref[1] + h_ref[...]) * dinv_ref[...] + b_ref[...]
    y_ref[...] = y
    rowid = lax.broadcasted_iota(jnp.int32, (R, 1), 0) + pid * R
    m = (rowid < nvalid).astype(jnp.float32)
    ym = y * m
    acc[0:1, :] += jnp.sum(ym, axis=0, keepdims=True)
    acc[1:2, :] += jnp.sum(ym * y, axis=0, keepdims=True)

    @pl.when(pid == pl.num_programs(0) - 1)
    def _():
        st_ref[...] = jnp.zeros_like(st_ref)
        st_ref[0:2, :] = acc[...]


def _norm_mm_body(nvalid, scale_out, y_ref, st_ref, g_ref, be_ref, w_ref,
                  dinv_ref, b1_ref, b2_ref, o_ref):
    inv_n = 1.0 / nvalid
    mu = st_ref[0:1, :] * inv_n
    var = st_ref[1:2, :] * inv_n - mu * mu
    istd = lax.rsqrt(var + 1e-5)
    xn = (y_ref[...] - mu) * istd * g_ref[...] + be_ref[...]
    r = jnp.maximum(xn, 0.0)
    h = lax.dot_general(r, w_ref[...], (((1,), (1,)), ((), ())),
                        preferred_element_type=jnp.float32)
    if scale_out:
        o_ref[...] = h * dinv_ref[...]
    else:
        o_ref[...] = h + b1_ref[...] + b2_ref[...]


def _lstm_body(C, H, z_ref, whh_ref, wo_ref, bo_ref, o_ref, h_s, c_s, hs_buf):
    pid = pl.program_id(0)

    @pl.when(pid == 0)
    def _():
        h_s[...] = jnp.zeros_like(h_s)
        c_s[...] = jnp.zeros_like(c_s)

    whh = whh_ref[...]

    def step(t, carry):
        h, c = carry
        zt = z_ref[pl.ds(t, 1), :]
        z = zt + lax.dot_general(h.astype(jnp.bfloat16), whh,
                                 (((1,), (1,)), ((), ())),
                                 preferred_element_type=jnp.float32)
        gi = jax.nn.sigmoid(z[:, 0:H])
        gf = jax.nn.sigmoid(z[:, H:2 * H])
        gg = jnp.tanh(z[:, 2 * H:3 * H])
        go = jax.nn.sigmoid(z[:, 3 * H:4 * H])
        c2 = gf * c + gi * gg
        h2 = go * jnp.tanh(c2)
        hs_buf[pl.ds(t, 1), :] = h2
        return (h2, c2)

    hf, cf = lax.fori_loop(0, C, step, (h_s[...], c_s[...]))
    h_s[...] = hf
    c_s[...] = cf
    out = lax.dot_general(hs_buf[...], wo_ref[...], (((1,), (1,)), ((), ())),
                          preferred_element_type=jnp.float32)
    o_ref[...] = out + bo_ref[...]


def kernel(x, edge_index, edge_weight, W1, b1, g1, be1, W2, b2, g2, be2,
           W_ih, W_hh, b_ih, b_hh, Wo, bo):
    N, D = x.shape
    H = W1.shape[0]
    E = edge_index.shape[1]
    P = Wo.shape[0]
    R = 1024
    NP = -(-N // R) * R
    EPU = _NCORES * _NTILES * _CHUNK * 12
    EP = -(-E // EPU) * EPU

    src = edge_index[0]
    dst = edge_index[1]
    pe = EP - E
    if pe:
        src = jnp.concatenate([src, jnp.zeros((pe,), src.dtype)])
        dst = jnp.concatenate([dst, jnp.zeros((pe,), dst.dtype)])
        ew = jnp.concatenate([edge_weight, jnp.zeros((pe,), edge_weight.dtype)])
    else:
        ew = edge_weight
    xp = jnp.pad(x, ((0, NP - N), (0, 0)))

    deg = _make_deg_dinv(NP, EP)(dst, ew)
    dinv = pl.pallas_call(
        _dinv_body,
        out_shape=jax.ShapeDtypeStruct((1, NP), jnp.float32),
    )(deg.reshape(_NCORES, NP))
    dinv_col = dinv.reshape(NP, 1)

    G = NP // R
    row_spec = pl.BlockSpec((R, H), lambda i: (i, 0))
    col_spec = pl.BlockSpec((R, 1), lambda i: (i, 0))
    w_spec = lambda shp: pl.BlockSpec(shp, lambda i: (0,) * len(shp))

    h1s = pl.pallas_call(
        _mm_scale_body,
        grid=(G,),
        in_specs=[pl.BlockSpec((R, D), lambda i: (i, 0)), w_spec((H, D)), col_spec],
        out_specs=row_spec,
        out_shape=jax.ShapeDtypeStruct((NP, H), jnp.float32),
    )(xp, W1, dinv_col)

    agg = _make_agg(NP, EP, H)

    def post(hs, p, b):
        return pl.pallas_call(
            functools.partial(_post_body, N, R),
            grid=(G,),
            in_specs=[row_spec,
                      pl.BlockSpec((2, R, H), lambda i: (0, i, 0)),
                      col_spec, w_spec((1, H))],
            out_specs=[row_spec, pl.BlockSpec((8, H), lambda i: (0, 0))],
            out_shape=[jax.ShapeDtypeStruct((NP, H), jnp.float32),
                       jax.ShapeDtypeStruct((8, H), jnp.float32)],
            scratch_shapes=[pltpu.VMEM((2, H), jnp.float32)],
        )(hs, p, dinv_col, b.reshape(1, H))

    def norm_mm(y, st, g, be, w, b1b, b2b, scale_out, KOUT):
        return pl.pallas_call(
            functools.partial(_norm_mm_body, N, scale_out),
            grid=(G,),
            in_specs=[row_spec, w_spec((8, H)), w_spec((1, H)), w_spec((1, H)),
                      w_spec((KOUT, H)), col_spec, w_spec((1, KOUT)),
                      w_spec((1, KOUT))],
            out_specs=pl.BlockSpec((R, KOUT), lambda i: (i, 0)),
            out_shape=jax.ShapeDtypeStruct((NP, KOUT), jnp.float32),
        )(y, st, g.reshape(1, H), be.reshape(1, H), w, dinv_col,
          b1b.reshape(1, KOUT), b2b.reshape(1, KOUT))

    p1 = agg(h1s, src, dst, ew)
    y1, st1 = post(h1s, p1, b1)
    zero_h = jnp.zeros((H,), jnp.float32)
    h2s = norm_mm(y1, st1, g1, be1, W2, zero_h, zero_h, True, H)
    p2 = agg(h2s, src, dst, ew)
    y2, st2 = post(h2s, p2, b2)
    zpre = norm_mm(y2, st2, g2, be2, W_ih, b_ih, b_hh, False, 4 * H)

    C = 1000 if N % 1000 == 0 else N
    GL = N // C
    out = pl.pallas_call(
        functools.partial(_lstm_body, C, H),
        grid=(GL,),
        in_specs=[pl.BlockSpec((C, 4 * H), lambda i: (i, 0)),
                  w_spec((4 * H, H)), w_spec((P, H)), w_spec((1, P))],
        out_specs=pl.BlockSpec((C, P), lambda i: (i, 0)),
        out_shape=jax.ShapeDtypeStruct((N, P), jnp.float32),
        scratch_shapes=[pltpu.VMEM((1, H), jnp.float32),
                        pltpu.VMEM((1, H), jnp.float32),
                        pltpu.VMEM((C, H), jnp.float32)],
    )(zpre, W_hh.astype(jnp.bfloat16), Wo, bo.reshape(1, P))
    return out


# speedup vs baseline: 16.3471x; 1.1934x over previous
import functools

import jax
import jax.numpy as jnp
from jax import lax
from jax.experimental import pallas as pl
from jax.experimental.pallas import tpu as pltpu
from jax.experimental.pallas import tpu_sc as plsc

_NCORES = 2
_NTILES = 16
_CHUNK = 120


def _make_deg_dinv(NP, EP):
    nch = EP // (_NCORES * _NTILES * _CHUNK)
    npt = NP // _NTILES
    mesh = plsc.VectorSubcoreMesh(core_axis_name="c", subcore_axis_name="s")

    @functools.partial(
        pl.kernel,
        out_type=jax.ShapeDtypeStruct((_NCORES * NP,), jnp.float32),
        mesh=mesh,
        compiler_params=pltpu.CompilerParams(needs_layout_passes=False),
        scratch_types=(
            [pltpu.VMEM((_CHUNK,), jnp.int32)] * 6
            + [pltpu.VMEM((_CHUNK,), jnp.float32)] * 6
            + [pltpu.VMEM((npt,), jnp.float32)]
            + [pltpu.VMEM_SHARED((NP,), jnp.float32)]
            + [pltpu.SemaphoreType.DMA] * 12
        ),
    )
    def deg_kernel(dst_hbm, w_hbm, deg_hbm, *refs):
        dstb = refs[0:6]
        wb = refs[6:12]
        dbuf = refs[12]
        acc = refs[13]
        esems = refs[14:20]
        ssems = refs[20:26]
        cid = lax.axis_index("c")
        sid = lax.axis_index("s")
        zv = jnp.zeros((16,), jnp.float32)

        def zb(i, _):
            dbuf[pl.ds(i * 16, 16)] = zv
            return 0

        lax.fori_loop(0, npt // 16, zb, 0)
        pltpu.sync_copy(dbuf, acc.at[pl.ds(sid * npt, npt)])
        plsc.subcore_barrier()

        tb = (cid * _NTILES + sid) * nch

        def ebase(j):
            return pl.multiple_of((tb + j) * _CHUNK, 8)

        def issue_e(j, e):
            pltpu.async_copy(dst_hbm.at[pl.ds(ebase(j), _CHUNK)], dstb[e],
                             esems[e])
            pltpu.async_copy(w_hbm.at[pl.ds(ebase(j), _CHUNK)], wb[e],
                             esems[e])

        def wait_e(j, e):
            pltpu.make_async_copy(dst_hbm.at[pl.ds(ebase(j), _CHUNK)],
                                  dstb[e], esems[e]).wait()
            pltpu.make_async_copy(w_hbm.at[pl.ds(ebase(j), _CHUNK)],
                                  wb[e], esems[e]).wait()

        def issue_s(j, e):
            pltpu.async_copy(wb[e], acc.at[dstb[e]], ssems[e], add=True)

        def wait_s(j, e):
            pltpu.make_async_copy(wb[e], acc.at[dstb[e]], ssems[e]).wait()

        for k in range(5):
            issue_e(k, k)

        def outer(jo, _):
            for b in range(6):
                j = jo * 6 + b
                wait_e(j, b)
                issue_s(j, b)

                @pl.when(j >= 1)
                def _():
                    wait_s(j - 1, (b + 5) % 6)

                @pl.when(j + 5 < nch)
                def _():
                    issue_e(j + 5, (b + 5) % 6)
            return 0

        lax.fori_loop(0, nch // 6, outer, 0)
        wait_s(nch - 1, (nch - 1) % 6)
        plsc.subcore_barrier()
        pltpu.sync_copy(acc.at[pl.ds(sid * npt, npt)],
                        deg_hbm.at[pl.ds(cid * NP + sid * npt, npt)])

    return deg_kernel


def _dinv_body(deg_ref, o_ref):
    o_ref[...] = lax.rsqrt(deg_ref[0:1, :] + deg_ref[1:2, :] + 1.0)


def _make_agg(NP, EP, F):
    nch = EP // (_NCORES * _NTILES * _CHUNK)
    npt = NP // _NTILES
    mesh = plsc.VectorSubcoreMesh(core_axis_name="c", subcore_axis_name="s")

    @functools.partial(
        pl.kernel,
        out_type=jax.ShapeDtypeStruct((_NCORES, NP, F), jnp.float32),
        mesh=mesh,
        compiler_params=pltpu.CompilerParams(needs_layout_passes=False),
        scratch_types=(
            [pltpu.VMEM((_CHUNK, F), jnp.float32)] * 3
            + [pltpu.VMEM((_CHUNK,), jnp.int32)] * 6
            + [pltpu.VMEM((_CHUNK,), jnp.int32)] * 6
            + [pltpu.VMEM((_CHUNK,), jnp.float32)] * 6
            + [pltpu.VMEM_SHARED((NP, F), jnp.float32)]
            + [pltpu.SemaphoreType.DMA] * 12
        ),
    )
    def agg_kernel(h_hbm, src_hbm, dst_hbm, w_hbm, out_hbm, *refs):
        bufs = refs[0:3]
        srcb = refs[3:9]
        dstb = refs[9:15]
        wb = refs[15:21]
        acc = refs[21]
        gsems = refs[22:25]
        ssems = refs[25:28]
        esems = refs[28:34]
        cid = lax.axis_index("c")
        sid = lax.axis_index("s")
        zv = jnp.zeros((16,), jnp.float32)

        def zb(r, _):
            for l in range(F // 16):
                bufs[0][r, pl.ds(l * 16, 16)] = zv
            return 0

        lax.fori_loop(0, _CHUNK, zb, 0)
        nfull, rem = npt // _CHUNK, npt % _CHUNK
        for i in range(nfull):
            pltpu.sync_copy(bufs[0],
                            acc.at[pl.ds(sid * npt + i * _CHUNK, _CHUNK)])
        if rem:
            pltpu.sync_copy(bufs[0].at[pl.ds(0, rem)],
                            acc.at[pl.ds(sid * npt + nfull * _CHUNK, rem)])
        plsc.subcore_barrier()

        tb = (cid * _NTILES + sid) * nch

        def ebase(j):
            return pl.multiple_of((tb + j) * _CHUNK, 8)

        def issue_e(j, e):
            pltpu.async_copy(src_hbm.at[pl.ds(ebase(j), _CHUNK)], srcb[e],
                             esems[e])
            pltpu.async_copy(dst_hbm.at[pl.ds(ebase(j), _CHUNK)], dstb[e],
                             esems[e])
            pltpu.async_copy(w_hbm.at[pl.ds(ebase(j), _CHUNK)], wb[e],
                             esems[e])

        def wait_e(j, e):
            pltpu.make_async_copy(src_hbm.at[pl.ds(ebase(j), _CHUNK)],
                                  srcb[e], esems[e]).wait()
            pltpu.make_async_copy(dst_hbm.at[pl.ds(ebase(j), _CHUNK)],
                                  dstb[e], esems[e]).wait()
            pltpu.make_async_copy(w_hbm.at[pl.ds(ebase(j), _CHUNK)],
                                  wb[e], esems[e]).wait()

        def issue_g(j, b, e):
            pltpu.async_copy(h_hbm.at[srcb[e]], bufs[b], gsems[b])

        def wait_g(j, b, e):
            pltpu.make_async_copy(h_hbm.at[srcb[e]], bufs[b], gsems[b]).wait()

        def issue_s(j, b, e):
            pltpu.async_copy(bufs[b], acc.at[dstb[e]], ssems[b], add=True)

        def wait_s(j, b, e):
            pltpu.make_async_copy(bufs[b], acc.at[dstb[e]], ssems[b]).wait()

        for k in range(5):
            issue_e(k, k)
        for k in range(2):
            wait_e(k, k)
            issue_g(k, k, k)

        def outer(jo, _):
            for b in range(6):
                j = jo * 6 + b
                bb = b % 3
                eb = b % 6
                wait_g(j, bb, eb)

                def scale(r4, _):
                    for u in range(4):
                        r = r4 * 4 + u
                        wr = plsc.load_gather(
                            wb[eb], [jnp.full((16,), r, jnp.int32)])
                        for l in range(F // 16):
                            bufs[bb][r, pl.ds(l * 16, 16)] = (
                                bufs[bb][r, pl.ds(l * 16, 16)] * wr)
                    return 0

                lax.fori_loop(0, _CHUNK // 4, scale, 0)
                issue_s(j, bb, eb)

                @pl.when(j >= 1)
                def _():
                    wait_s(j - 1, (bb + 2) % 3, (eb + 5) % 6)

                @pl.when(j + 5 < nch)
                def _():
                    issue_e(j + 5, (eb + 5) % 6)

                @pl.when(j + 2 < nch)
                def _():
                    wait_e(j + 2, (eb + 2) % 6)
                    issue_g(j + 2, (bb + 2) % 3, (eb + 2) % 6)
            return 0

        lax.fori_loop(0, nch // 6, outer, 0)
        wait_s(nch - 1, (nch - 1) % 3, (nch - 1) % 6)
        plsc.subcore_barrier()
        pltpu.sync_copy(acc.at[pl.ds(sid * npt, npt)],
                        out_hbm.at[cid, pl.ds(sid * npt, npt)])

    return agg_kernel


def _mm_scale_body(x_ref, w_ref, dinv_ref, o_ref):
    h = lax.dot_general(x_ref[...], w_ref[...], (((1,), (1,)), ((), ())),
                        preferred_element_type=jnp.float32)
    o_ref[...] = h * dinv_ref[...]


def _post_body(nvalid, R, h_ref, p_ref, dinv_ref, b_ref, y_ref, st_ref, acc):
    pid = pl.program_id(0)

    @pl.when(pid == 0)
    def _():
        acc[...] = jnp.zeros_like(acc)

    y = (p_ref[0] + p_ref[1] + h_ref[...]) * dinv_ref[...] + b_ref[...]
    y_ref[...] = y
    rowid = lax.broadcasted_iota(jnp.int32, (R, 1), 0) + pid * R
    m = (rowid < nvalid).astype(jnp.float32)
    ym = y * m
    acc[0:1, :] += jnp.sum(ym, axis=0, keepdims=True)
    acc[1:2, :] += jnp.sum(ym * y, axis=0, keepdims=True)

    @pl.when(pid == pl.num_programs(0) - 1)
    def _():
        st_ref[...] = jnp.zeros_like(st_ref)
        st_ref[0:2, :] = acc[...]


def _norm_mm_body(nvalid, scale_out, y_ref, st_ref, g_ref, be_ref, w_ref,
                  dinv_ref, b1_ref, b2_ref, o_ref):
    inv_n = 1.0 / nvalid
    mu = st_ref[0:1, :] * inv_n
    var = st_ref[1:2, :] * inv_n - mu * mu
    istd = lax.rsqrt(var + 1e-5)
    xn = (y_ref[...] - mu) * istd * g_ref[...] + be_ref[...]
    r = jnp.maximum(xn, 0.0)
    h = lax.dot_general(r, w_ref[...], (((1,), (1,)), ((), ())),
                        preferred_element_type=jnp.float32)
    if scale_out:
        o_ref[...] = h * dinv_ref[...]
    else:
        o_ref[...] = h + b1_ref[...] + b2_ref[...]


def _lstm_body(C, H, z_ref, whh_ref, wo_ref, bo_ref, o_ref, h_s, c_s, hs_buf):
    pid = pl.program_id(0)

    @pl.when(pid == 0)
    def _():
        h_s[...] = jnp.zeros_like(h_s)
        c_s[...] = jnp.zeros_like(c_s)

    whh = whh_ref[...]

    def step(t, carry):
        hcol, c = carry
        zt = z_ref[pl.ds(t, 1), :]
        z = zt + jnp.sum(whh * hcol, axis=0, keepdims=True)
        gi = jax.nn.sigmoid(z[:, 0:H])
        gf = jax.nn.sigmoid(z[:, H:2 * H])
        gg = jnp.tanh(z[:, 2 * H:3 * H])
        go = jax.nn.sigmoid(z[:, 3 * H:4 * H])
        c2 = gf * c + gi * gg
        h2 = go * jnp.tanh(c2)
        hs_buf[pl.ds(t, 1), :] = h2
        return (h2.reshape(H, 1), c2)

    hf, cf = lax.fori_loop(0, C, step, (h_s[...].reshape(H, 1), c_s[...]))
    h_s[...] = hf.reshape(1, H)
    c_s[...] = cf
    out = lax.dot_general(hs_buf[...], wo_ref[...], (((1,), (1,)), ((), ())),
                          preferred_element_type=jnp.float32)
    o_ref[...] = out + bo_ref[...]


def kernel(x, edge_index, edge_weight, W1, b1, g1, be1, W2, b2, g2, be2,
           W_ih, W_hh, b_ih, b_hh, Wo, bo):
    N, D = x.shape
    H = W1.shape[0]
    E = edge_index.shape[1]
    P = Wo.shape[0]
    R = 1024
    NP = -(-N // R) * R
    EPU = _NCORES * _NTILES * _CHUNK * 12
    EP = -(-E // EPU) * EPU

    src = edge_index[0]
    dst = edge_index[1]
    pe = EP - E
    if pe:
        src = jnp.concatenate([src, jnp.zeros((pe,), src.dtype)])
        dst = jnp.concatenate([dst, jnp.zeros((pe,), dst.dtype)])
        ew = jnp.concatenate([edge_weight, jnp.zeros((pe,), edge_weight.dtype)])
    else:
        ew = edge_weight
    xp = jnp.pad(x, ((0, NP - N), (0, 0)))

    deg = _make_deg_dinv(NP, EP)(dst, ew)
    dinv = pl.pallas_call(
        _dinv_body,
        out_shape=jax.ShapeDtypeStruct((1, NP), jnp.float32),
    )(deg.reshape(_NCORES, NP))
    dinv_col = dinv.reshape(NP, 1)

    G = NP // R
    row_spec = pl.BlockSpec((R, H), lambda i: (i, 0))
    col_spec = pl.BlockSpec((R, 1), lambda i: (i, 0))
    w_spec = lambda shp: pl.BlockSpec(shp, lambda i: (0,) * len(shp))

    h1s = pl.pallas_call(
        _mm_scale_body,
        grid=(G,),
        in_specs=[pl.BlockSpec((R, D), lambda i: (i, 0)), w_spec((H, D)), col_spec],
        out_specs=row_spec,
        out_shape=jax.ShapeDtypeStruct((NP, H), jnp.float32),
    )(xp, W1, dinv_col)

    agg = _make_agg(NP, EP, H)

    def post(hs, p, b):
        return pl.pallas_call(
            functools.partial(_post_body, N, R),
            grid=(G,),
            in_specs=[row_spec,
                      pl.BlockSpec((2, R, H), lambda i: (0, i, 0)),
                      col_spec, w_spec((1, H))],
            out_specs=[row_spec, pl.BlockSpec((8, H), lambda i: (0, 0))],
            out_shape=[jax.ShapeDtypeStruct((NP, H), jnp.float32),
                       jax.ShapeDtypeStruct((8, H), jnp.float32)],
            scratch_shapes=[pltpu.VMEM((2, H), jnp.float32)],
        )(hs, p, dinv_col, b.reshape(1, H))

    def norm_mm(y, st, g, be, w, b1b, b2b, scale_out, KOUT):
        return pl.pallas_call(
            functools.partial(_norm_mm_body, N, scale_out),
            grid=(G,),
            in_specs=[row_spec, w_spec((8, H)), w_spec((1, H)), w_spec((1, H)),
                      w_spec((KOUT, H)), col_spec, w_spec((1, KOUT)),
                      w_spec((1, KOUT))],
            out_specs=pl.BlockSpec((R, KOUT), lambda i: (i, 0)),
            out_shape=jax.ShapeDtypeStruct((NP, KOUT), jnp.float32),
        )(y, st, g.reshape(1, H), be.reshape(1, H), w, dinv_col,
          b1b.reshape(1, KOUT), b2b.reshape(1, KOUT))

    p1 = agg(h1s, src, dst, ew)
    y1, st1 = post(h1s, p1, b1)
    zero_h = jnp.zeros((H,), jnp.float32)
    h2s = norm_mm(y1, st1, g1, be1, W2, zero_h, zero_h, True, H)
    p2 = agg(h2s, src, dst, ew)
    y2, st2 = post(h2s, p2, b2)
    zpre = norm_mm(y2, st2, g2, be2, W_ih, b_ih, b_hh, False, 4 * H)

    C = 1000 if N % 1000 == 0 else N
    GL = N // C
    out = pl.pallas_call(
        functools.partial(_lstm_body, C, H),
        grid=(GL,),
        in_specs=[pl.BlockSpec((C, 4 * H), lambda i: (i, 0)),
                  w_spec((H, 4 * H)), w_spec((P, H)), w_spec((1, P))],
        out_specs=pl.BlockSpec((C, P), lambda i: (i, 0)),
        out_shape=jax.ShapeDtypeStruct((N, P), jnp.float32),
        scratch_shapes=[pltpu.VMEM((1, H), jnp.float32),
                        pltpu.VMEM((1, H), jnp.float32),
                        pltpu.VMEM((C, H), jnp.float32)],
    )(zpre, W_hh.T, Wo, bo.reshape(1, P))
    return out


# speedup vs baseline: 18.2491x; 1.1164x over previous
import functools

import jax
import jax.numpy as jnp
from jax import lax
from jax.experimental import pallas as pl
from jax.experimental.pallas import tpu as pltpu
from jax.experimental.pallas import tpu_sc as plsc

_NCORES = 2
_NTILES = 16
_CHUNK = 120


def _make_deg_dinv(NP, EP):
    nch = EP // (_NCORES * _NTILES * _CHUNK)
    npt = NP // _NTILES
    mesh = plsc.VectorSubcoreMesh(core_axis_name="c", subcore_axis_name="s")

    @functools.partial(
        pl.kernel,
        out_type=jax.ShapeDtypeStruct((_NCORES * NP,), jnp.float32),
        mesh=mesh,
        compiler_params=pltpu.CompilerParams(needs_layout_passes=False),
        scratch_types=(
            [pltpu.VMEM((_CHUNK,), jnp.int32)] * 6
            + [pltpu.VMEM((_CHUNK,), jnp.float32)] * 6
            + [pltpu.VMEM((npt,), jnp.float32)]
            + [pltpu.VMEM_SHARED((NP,), jnp.float32)]
            + [pltpu.SemaphoreType.DMA] * 12
        ),
    )
    def deg_kernel(dst_hbm, w_hbm, deg_hbm, *refs):
        dstb = refs[0:6]
        wb = refs[6:12]
        dbuf = refs[12]
        acc = refs[13]
        esems = refs[14:20]
        ssems = refs[20:26]
        cid = lax.axis_index("c")
        sid = lax.axis_index("s")
        zv = jnp.zeros((16,), jnp.float32)

        def zb(i, _):
            dbuf[pl.ds(i * 16, 16)] = zv
            return 0

        lax.fori_loop(0, npt // 16, zb, 0)
        pltpu.sync_copy(dbuf, acc.at[pl.ds(sid * npt, npt)])
        plsc.subcore_barrier()

        tb = (cid * _NTILES + sid) * nch

        def ebase(j):
            return pl.multiple_of((tb + j) * _CHUNK, 8)

        def issue_e(j, e):
            pltpu.async_copy(dst_hbm.at[pl.ds(ebase(j), _CHUNK)], dstb[e],
                             esems[e])
            pltpu.async_copy(w_hbm.at[pl.ds(ebase(j), _CHUNK)], wb[e],
                             esems[e])

        def wait_e(j, e):
            pltpu.make_async_copy(dst_hbm.at[pl.ds(ebase(j), _CHUNK)],
                                  dstb[e], esems[e]).wait()
            pltpu.make_async_copy(w_hbm.at[pl.ds(ebase(j), _CHUNK)],
                                  wb[e], esems[e]).wait()

        def issue_s(j, e):
            pltpu.async_copy(wb[e], acc.at[dstb[e]], ssems[e], add=True)

        def wait_s(j, e):
            pltpu.make_async_copy(wb[e], acc.at[dstb[e]], ssems[e]).wait()

        for k in range(5):
            issue_e(k, k)

        def outer(jo, _):
            for b in range(6):
                j = jo * 6 + b
                wait_e(j, b)
                issue_s(j, b)

                @pl.when(j >= 1)
                def _():
                    wait_s(j - 1, (b + 5) % 6)

                @pl.when(j + 5 < nch)
                def _():
                    issue_e(j + 5, (b + 5) % 6)
            return 0

        lax.fori_loop(0, nch // 6, outer, 0)
        wait_s(nch - 1, (nch - 1) % 6)
        plsc.subcore_barrier()
        pltpu.sync_copy(acc.at[pl.ds(sid * npt, npt)],
                        deg_hbm.at[pl.ds(cid * NP + sid * npt, npt)])

    return deg_kernel


def _dinv_body(deg_ref, o_ref):
    o_ref[...] = lax.rsqrt(deg_ref[0:1, :] + deg_ref[1:2, :] + 1.0)


def _make_agg(NP, EP, F):
    nch = EP // (_NCORES * _NTILES * _CHUNK)
    npt = NP // _NTILES
    mesh = plsc.VectorSubcoreMesh(core_axis_name="c", subcore_axis_name="s")

    @functools.partial(
        pl.kernel,
        out_type=jax.ShapeDtypeStruct((_NCORES, NP, F), jnp.float32),
        mesh=mesh,
        compiler_params=pltpu.CompilerParams(needs_layout_passes=False),
        scratch_types=(
            [pltpu.VMEM((_CHUNK, F), jnp.float32)] * 3
            + [pltpu.VMEM((_CHUNK,), jnp.int32)] * 6
            + [pltpu.VMEM((_CHUNK,), jnp.int32)] * 6
            + [pltpu.VMEM((_CHUNK,), jnp.float32)] * 6
            + [pltpu.VMEM_SHARED((NP, F), jnp.float32)]
            + [pltpu.SemaphoreType.DMA] * 12
        ),
    )
    def agg_kernel(h_hbm, src_hbm, dst_hbm, w_hbm, out_hbm, *refs):
        bufs = refs[0:3]
        srcb = refs[3:9]
        dstb = refs[9:15]
        wb = refs[15:21]
        acc = refs[21]
        gsems = refs[22:25]
        ssems = refs[25:28]
        esems = refs[28:34]
        cid = lax.axis_index("c")
        sid = lax.axis_index("s")
        zv = jnp.zeros((16,), jnp.float32)

        def zb(r, _):
            for l in range(F // 16):
                bufs[0][r, pl.ds(l * 16, 16)] = zv
            return 0

        lax.fori_loop(0, _CHUNK, zb, 0)
        nfull, rem = npt // _CHUNK, npt % _CHUNK
        for i in range(nfull):
            pltpu.sync_copy(bufs[0],
                            acc.at[pl.ds(sid * npt + i * _CHUNK, _CHUNK)])
        if rem:
            pltpu.sync_copy(bufs[0].at[pl.ds(0, rem)],
                            acc.at[pl.ds(sid * npt + nfull * _CHUNK, rem)])
        plsc.subcore_barrier()

        tb = (cid * _NTILES + sid) * nch

        def ebase(j):
            return pl.multiple_of((tb + j) * _CHUNK, 8)

        def issue_e(j, e):
            pltpu.async_copy(src_hbm.at[pl.ds(ebase(j), _CHUNK)], srcb[e],
                             esems[e])
            pltpu.async_copy(dst_hbm.at[pl.ds(ebase(j), _CHUNK)], dstb[e],
                             esems[e])
            pltpu.async_copy(w_hbm.at[pl.ds(ebase(j), _CHUNK)], wb[e],
                             esems[e])

        def wait_e(j, e):
            pltpu.make_async_copy(src_hbm.at[pl.ds(ebase(j), _CHUNK)],
                                  srcb[e], esems[e]).wait()
            pltpu.make_async_copy(dst_hbm.at[pl.ds(ebase(j), _CHUNK)],
                                  dstb[e], esems[e]).wait()
            pltpu.make_async_copy(w_hbm.at[pl.ds(ebase(j), _CHUNK)],
                                  wb[e], esems[e]).wait()

        def issue_g(j, b, e):
            pltpu.async_copy(h_hbm.at[srcb[e]], bufs[b], gsems[b])

        def wait_g(j, b, e):
            pltpu.make_async_copy(h_hbm.at[srcb[e]], bufs[b], gsems[b]).wait()

        def issue_s(j, b, e):
            pltpu.async_copy(bufs[b], acc.at[dstb[e]], ssems[b], add=True)

        def wait_s(j, b, e):
            pltpu.make_async_copy(bufs[b], acc.at[dstb[e]], ssems[b]).wait()

        for k in range(5):
            issue_e(k, k)
        for k in range(2):
            wait_e(k, k)
            issue_g(k, k, k)

        def outer(jo, _):
            for b in range(6):
                j = jo * 6 + b
                bb = b % 3
                eb = b % 6
                wait_g(j, bb, eb)

                def scale(r4, _):
                    for u in range(4):
                        r = r4 * 4 + u
                        wr = plsc.load_gather(
                            wb[eb], [jnp.full((16,), r, jnp.int32)])
                        for l in range(F // 16):
                            bufs[bb][r, pl.ds(l * 16, 16)] = (
                                bufs[bb][r, pl.ds(l * 16, 16)] * wr)
                    return 0

                lax.fori_loop(0, _CHUNK // 4, scale, 0)
                issue_s(j, bb, eb)

                @pl.when(j >= 1)
                def _():
                    wait_s(j - 1, (bb + 2) % 3, (eb + 5) % 6)

                @pl.when(j + 5 < nch)
                def _():
                    issue_e(j + 5, (eb + 5) % 6)

                @pl.when(j + 2 < nch)
                def _():
                    wait_e(j + 2, (eb + 2) % 6)
                    issue_g(j + 2, (bb + 2) % 3, (eb + 2) % 6)
            return 0

        lax.fori_loop(0, nch // 6, outer, 0)
        wait_s(nch - 1, (nch - 1) % 3, (nch - 1) % 6)
        plsc.subcore_barrier()
        pltpu.sync_copy(acc.at[pl.ds(sid * npt, npt)],
                        out_hbm.at[cid, pl.ds(sid * npt, npt)])

    return agg_kernel


def _mm_scale_body(x_ref, w_ref, dinv_ref, o_ref):
    h = lax.dot_general(x_ref[...], w_ref[...], (((1,), (1,)), ((), ())),
                        preferred_element_type=jnp.float32)
    o_ref[...] = h * dinv_ref[...]


def _post_body(nvalid, R, h_ref, p_ref, dinv_ref, b_ref, y_ref, st_ref, acc):
    pid = pl.program_id(0)

    @pl.when(pid == 0)
    def _():
        acc[...] = jnp.zeros_like(acc)

    y = (p_ref[0] + p_ref[1] + h_ref[...]) * dinv_ref[...] + b_ref[...]
    y_ref[...] = y
    rowid = lax.broadcasted_iota(jnp.int32, (R, 1), 0) + pid * R
    m = (rowid < nvalid).astype(jnp.float32)
    ym = y * m
    acc[0:1, :] += jnp.sum(ym, axis=0, keepdims=True)
    acc[1:2, :] += jnp.sum(ym * y, axis=0, keepdims=True)

    @pl.when(pid == pl.num_programs(0) - 1)
    def _():
        st_ref[...] = jnp.zeros_like(st_ref)
        st_ref[0:2, :] = acc[...]


def _norm_mm_body(nvalid, scale_out, y_ref, st_ref, g_ref, be_ref, w_ref,
                  dinv_ref, b1_ref, b2_ref, o_ref):
    inv_n = 1.0 / nvalid
    mu = st_ref[0:1, :] * inv_n
    var = st_ref[1:2, :] * inv_n - mu * mu
    istd = lax.rsqrt(var + 1e-5)
    xn = (y_ref[...] - mu) * istd * g_ref[...] + be_ref[...]
    r = jnp.maximum(xn, 0.0)
    h = lax.dot_general(r, w_ref[...], (((1,), (1,)), ((), ())),
                        preferred_element_type=jnp.float32)
    if scale_out:
        o_ref[...] = h * dinv_ref[...]
    else:
        o_ref[...] = h + b1_ref[...] + b2_ref[...]


def _lstm_body(C, H, z_ref, whh_ref, wo_ref, bo_ref, o_ref, h_s, c_s, hs_buf):
    pid = pl.program_id(0)

    @pl.when(pid == 0)
    def _():
        h_s[...] = jnp.zeros_like(h_s)
        c_s[...] = jnp.zeros_like(c_s)

    whh = whh_ref[...]

    def step2(i, carry):
        h, c = carry
        for u in range(2):
            t = i * 2 + u
            zt = z_ref[pl.ds(t, 1), :]
            z = zt + jnp.sum(whh * h.reshape(H, 1), axis=0, keepdims=True)
            gi = 0.5 + 0.5 * jnp.tanh(0.5 * z[:, 0:H])
            gf = 0.5 + 0.5 * jnp.tanh(0.5 * z[:, H:2 * H])
            gg = jnp.tanh(z[:, 2 * H:3 * H])
            go = 0.5 + 0.5 * jnp.tanh(0.5 * z[:, 3 * H:4 * H])
            c = gf * c + gi * gg
            h = go * jnp.tanh(c)
            hs_buf[pl.ds(t, 1), :] = h
        return (h, c)

    hf, cf = lax.fori_loop(0, C // 2, step2, (h_s[...], c_s[...]))
    h_s[...] = hf
    c_s[...] = cf
    out = lax.dot_general(hs_buf[...], wo_ref[...], (((1,), (1,)), ((), ())),
                          preferred_element_type=jnp.float32)
    o_ref[...] = out + bo_ref[...]


def kernel(x, edge_index, edge_weight, W1, b1, g1, be1, W2, b2, g2, be2,
           W_ih, W_hh, b_ih, b_hh, Wo, bo):
    N, D = x.shape
    H = W1.shape[0]
    E = edge_index.shape[1]
    P = Wo.shape[0]
    R = 1024
    NP = -(-N // R) * R
    EPU = _NCORES * _NTILES * _CHUNK * 12
    EP = -(-E // EPU) * EPU

    src = edge_index[0]
    dst = edge_index[1]
    pe = EP - E
    if pe:
        src = jnp.concatenate([src, jnp.zeros((pe,), src.dtype)])
        dst = jnp.concatenate([dst, jnp.zeros((pe,), dst.dtype)])
        ew = jnp.concatenate([edge_weight, jnp.zeros((pe,), edge_weight.dtype)])
    else:
        ew = edge_weight
    xp = jnp.pad(x, ((0, NP - N), (0, 0)))

    deg = _make_deg_dinv(NP, EP)(dst, ew)
    dinv = pl.pallas_call(
        _dinv_body,
        out_shape=jax.ShapeDtypeStruct((1, NP), jnp.float32),
    )(deg.reshape(_NCORES, NP))
    dinv_col = dinv.reshape(NP, 1)

    G = NP // R
    row_spec = pl.BlockSpec((R, H), lambda i: (i, 0))
    col_spec = pl.BlockSpec((R, 1), lambda i: (i, 0))
    w_spec = lambda shp: pl.BlockSpec(shp, lambda i: (0,) * len(shp))

    h1s = pl.pallas_call(
        _mm_scale_body,
        grid=(G,),
        in_specs=[pl.BlockSpec((R, D), lambda i: (i, 0)), w_spec((H, D)), col_spec],
        out_specs=row_spec,
        out_shape=jax.ShapeDtypeStruct((NP, H), jnp.float32),
    )(xp, W1, dinv_col)

    agg = _make_agg(NP, EP, H)

    def post(hs, p, b):
        return pl.pallas_call(
            functools.partial(_post_body, N, R),
            grid=(G,),
            in_specs=[row_spec,
                      pl.BlockSpec((2, R, H), lambda i: (0, i, 0)),
                      col_spec, w_spec((1, H))],
            out_specs=[row_spec, pl.BlockSpec((8, H), lambda i: (0, 0))],
            out_shape=[jax.ShapeDtypeStruct((NP, H), jnp.float32),
                       jax.ShapeDtypeStruct((8, H), jnp.float32)],
            scratch_shapes=[pltpu.VMEM((2, H), jnp.float32)],
        )(hs, p, dinv_col, b.reshape(1, H))

    def norm_mm(y, st, g, be, w, b1b, b2b, scale_out, KOUT):
        return pl.pallas_call(
            functools.partial(_norm_mm_body, N, scale_out),
            grid=(G,),
            in_specs=[row_spec, w_spec((8, H)), w_spec((1, H)), w_spec((1, H)),
                      w_spec((KOUT, H)), col_spec, w_spec((1, KOUT)),
                      w_spec((1, KOUT))],
            out_specs=pl.BlockSpec((R, KOUT), lambda i: (i, 0)),
            out_shape=jax.ShapeDtypeStruct((NP, KOUT), jnp.float32),
        )(y, st, g.reshape(1, H), be.reshape(1, H), w, dinv_col,
          b1b.reshape(1, KOUT), b2b.reshape(1, KOUT))

    p1 = agg(h1s, src, dst, ew)
    y1, st1 = post(h1s, p1, b1)
    zero_h = jnp.zeros((H,), jnp.float32)
    h2s = norm_mm(y1, st1, g1, be1, W2, zero_h, zero_h, True, H)
    p2 = agg(h2s, src, dst, ew)
    y2, st2 = post(h2s, p2, b2)
    zpre = norm_mm(y2, st2, g2, be2, W_ih, b_ih, b_hh, False, 4 * H)

    C = 1000 if N % 1000 == 0 else N
    GL = N // C
    out = pl.pallas_call(
        functools.partial(_lstm_body, C, H),
        grid=(GL,),
        in_specs=[pl.BlockSpec((C, 4 * H), lambda i: (i, 0)),
                  w_spec((H, 4 * H)), w_spec((P, H)), w_spec((1, P))],
        out_specs=pl.BlockSpec((C, P), lambda i: (i, 0)),
        out_shape=jax.ShapeDtypeStruct((N, P), jnp.float32),
        scratch_shapes=[pltpu.VMEM((1, H), jnp.float32),
                        pltpu.VMEM((1, H), jnp.float32),
                        pltpu.VMEM((C, H), jnp.float32)],
    )(zpre, W_hh.T, Wo, bo.reshape(1, P))
    return out


# speedup vs baseline: 18.2649x; 1.0009x over previous
import functools

import jax
import jax.numpy as jnp
from jax import lax
from jax.experimental import pallas as pl
from jax.experimental.pallas import tpu as pltpu
from jax.experimental.pallas import tpu_sc as plsc

_NCORES = 2
_NTILES = 16
_CHUNK = 120


def _make_deg_dinv(NP, EP):
    nch = EP // (_NCORES * _NTILES * _CHUNK)
    npt = NP // _NTILES
    mesh = plsc.VectorSubcoreMesh(core_axis_name="c", subcore_axis_name="s")

    @functools.partial(
        pl.kernel,
        out_type=jax.ShapeDtypeStruct((_NCORES * NP,), jnp.float32),
        mesh=mesh,
        compiler_params=pltpu.CompilerParams(needs_layout_passes=False),
        scratch_types=(
            [pltpu.VMEM((_CHUNK,), jnp.int32)] * 6
            + [pltpu.VMEM((_CHUNK,), jnp.float32)] * 6
            + [pltpu.VMEM((npt,), jnp.float32)]
            + [pltpu.VMEM_SHARED((NP,), jnp.float32)]
            + [pltpu.SemaphoreType.DMA] * 12
        ),
    )
    def deg_kernel(dst_hbm, w_hbm, deg_hbm, *refs):
        dstb = refs[0:6]
        wb = refs[6:12]
        dbuf = refs[12]
        acc = refs[13]
        esems = refs[14:20]
        ssems = refs[20:26]
        cid = lax.axis_index("c")
        sid = lax.axis_index("s")
        zv = jnp.zeros((16,), jnp.float32)

        def zb(i, _):
            dbuf[pl.ds(i * 16, 16)] = zv
            return 0

        lax.fori_loop(0, npt // 16, zb, 0)
        pltpu.sync_copy(dbuf, acc.at[pl.ds(sid * npt, npt)])
        plsc.subcore_barrier()

        tb = (cid * _NTILES + sid) * nch

        def ebase(j):
            return pl.multiple_of((tb + j) * _CHUNK, 8)

        def issue_e(j, e):
            pltpu.async_copy(dst_hbm.at[pl.ds(ebase(j), _CHUNK)], dstb[e],
                             esems[e])
            pltpu.async_copy(w_hbm.at[pl.ds(ebase(j), _CHUNK)], wb[e],
                             esems[e])

        def wait_e(j, e):
            pltpu.make_async_copy(dst_hbm.at[pl.ds(ebase(j), _CHUNK)],
                                  dstb[e], esems[e]).wait()
            pltpu.make_async_copy(w_hbm.at[pl.ds(ebase(j), _CHUNK)],
                                  wb[e], esems[e]).wait()

        def issue_s(j, e):
            pltpu.async_copy(wb[e], acc.at[dstb[e]], ssems[e], add=True)

        def wait_s(j, e):
            pltpu.make_async_copy(wb[e], acc.at[dstb[e]], ssems[e]).wait()

        for k in range(5):
            issue_e(k, k)

        def outer(jo, _):
            for b in range(6):
                j = jo * 6 + b
                wait_e(j, b)
                issue_s(j, b)

                @pl.when(j >= 1)
                def _():
                    wait_s(j - 1, (b + 5) % 6)

                @pl.when(j + 5 < nch)
                def _():
                    issue_e(j + 5, (b + 5) % 6)
            return 0

        lax.fori_loop(0, nch // 6, outer, 0)
        wait_s(nch - 1, (nch - 1) % 6)
        plsc.subcore_barrier()
        pltpu.sync_copy(acc.at[pl.ds(sid * npt, npt)],
                        deg_hbm.at[pl.ds(cid * NP + sid * npt, npt)])

    return deg_kernel


def _dinv_body(deg_ref, o_ref):
    o_ref[...] = lax.rsqrt(deg_ref[0:1, :] + deg_ref[1:2, :] + 1.0)


def _make_agg(NP, EP, F):
    nch = EP // (_NCORES * _NTILES * _CHUNK)
    npt = NP // _NTILES
    mesh = plsc.VectorSubcoreMesh(core_axis_name="c", subcore_axis_name="s")

    @functools.partial(
        pl.kernel,
        out_type=jax.ShapeDtypeStruct((_NCORES, NP, F), jnp.float32),
        mesh=mesh,
        compiler_params=pltpu.CompilerParams(needs_layout_passes=False),
        scratch_types=(
            [pltpu.VMEM((_CHUNK, F), jnp.float32)] * 3
            + [pltpu.VMEM((_CHUNK,), jnp.int32)] * 6
            + [pltpu.VMEM((_CHUNK,), jnp.int32)] * 6
            + [pltpu.VMEM((_CHUNK,), jnp.float32)] * 6
            + [pltpu.VMEM_SHARED((NP, F), jnp.float32)]
            + [pltpu.SemaphoreType.DMA] * 12
        ),
    )
    def agg_kernel(h_hbm, src_hbm, dst_hbm, w_hbm, out_hbm, *refs):
        bufs = refs[0:3]
        srcb = refs[3:9]
        dstb = refs[9:15]
        wb = refs[15:21]
        acc = refs[21]
        gsems = refs[22:25]
        ssems = refs[25:28]
        esems = refs[28:34]
        cid = lax.axis_index("c")
        sid = lax.axis_index("s")
        zv = jnp.zeros((16,), jnp.float32)

        def zb(r, _):
            for l in range(F // 16):
                bufs[0][r, pl.ds(l * 16, 16)] = zv
            return 0

        lax.fori_loop(0, _CHUNK, zb, 0)
        nfull, rem = npt // _CHUNK, npt % _CHUNK
        for i in range(nfull):
            pltpu.sync_copy(bufs[0],
                            acc.at[pl.ds(sid * npt + i * _CHUNK, _CHUNK)])
        if rem:
            pltpu.sync_copy(bufs[0].at[pl.ds(0, rem)],
                            acc.at[pl.ds(sid * npt + nfull * _CHUNK, rem)])
        plsc.subcore_barrier()

        tb = (cid * _NTILES + sid) * nch

        def ebase(j):
            return pl.multiple_of((tb + j) * _CHUNK, 8)

        def issue_e(j, e):
            pltpu.async_copy(src_hbm.at[pl.ds(ebase(j), _CHUNK)], srcb[e],
                             esems[e])
            pltpu.async_copy(dst_hbm.at[pl.ds(ebase(j), _CHUNK)], dstb[e],
                             esems[e])
            pltpu.async_copy(w_hbm.at[pl.ds(ebase(j), _CHUNK)], wb[e],
                             esems[e])

        def wait_e(j, e):
            pltpu.make_async_copy(src_hbm.at[pl.ds(ebase(j), _CHUNK)],
                                  srcb[e], esems[e]).wait()
            pltpu.make_async_copy(dst_hbm.at[pl.ds(ebase(j), _CHUNK)],
                                  dstb[e], esems[e]).wait()
            pltpu.make_async_copy(w_hbm.at[pl.ds(ebase(j), _CHUNK)],
                                  wb[e], esems[e]).wait()

        def issue_g(j, b, e):
            pltpu.async_copy(h_hbm.at[srcb[e]], bufs[b], gsems[b])

        def wait_g(j, b, e):
            pltpu.make_async_copy(h_hbm.at[srcb[e]], bufs[b], gsems[b]).wait()

        def issue_s(j, b, e):
            pltpu.async_copy(bufs[b], acc.at[dstb[e]], ssems[b], add=True)

        def wait_s(j, b, e):
            pltpu.make_async_copy(bufs[b], acc.at[dstb[e]], ssems[b]).wait()

        for k in range(5):
            issue_e(k, k)
        for k in range(2):
            wait_e(k, k)
            issue_g(k, k, k)

        def outer(jo, _):
            for b in range(6):
                j = jo * 6 + b
                bb = b % 3
                eb = b % 6
                wait_g(j, bb, eb)

                def scale(r4, _):
                    for u in range(4):
                        r = r4 * 4 + u
                        wr = plsc.load_gather(
                            wb[eb], [jnp.full((16,), r, jnp.int32)])
                        for l in range(F // 16):
                            bufs[bb][r, pl.ds(l * 16, 16)] = (
                                bufs[bb][r, pl.ds(l * 16, 16)] * wr)
                    return 0

                lax.fori_loop(0, _CHUNK // 4, scale, 0)
                issue_s(j, bb, eb)

                @pl.when(j >= 1)
                def _():
                    wait_s(j - 1, (bb + 2) % 3, (eb + 5) % 6)

                @pl.when(j + 5 < nch)
                def _():
                    issue_e(j + 5, (eb + 5) % 6)

                @pl.when(j + 2 < nch)
                def _():
                    wait_e(j + 2, (eb + 2) % 6)
                    issue_g(j + 2, (bb + 2) % 3, (eb + 2) % 6)
            return 0

        lax.fori_loop(0, nch // 6, outer, 0)
        wait_s(nch - 1, (nch - 1) % 3, (nch - 1) % 6)
        plsc.subcore_barrier()
        pltpu.sync_copy(acc.at[pl.ds(sid * npt, npt)],
                        out_hbm.at[cid, pl.ds(sid * npt, npt)])

    return agg_kernel


def _mm_scale_body(x_ref, w_ref, dinv_ref, o_ref):
    h = lax.dot_general(x_ref[...], w_ref[...], (((1,), (1,)), ((), ())),
                        preferred_element_type=jnp.float32)
    o_ref[...] = h * dinv_ref[...]


def _post_body(nvalid, R, h_ref, p_ref, dinv_ref, b_ref, y_ref, st_ref, acc):
    pid = pl.program_id(0)

    @pl.when(pid == 0)
    def _():
        acc[...] = jnp.zeros_like(acc)

    y = (p_ref[0] + p_ref[1] + h_ref[...]) * dinv_ref[...] + b_ref[...]
    y_ref[...] = y
    rowid = lax.broadcasted_iota(jnp.int32, (R, 1), 0) + pid * R
    m = (rowid < nvalid).astype(jnp.float32)
    ym = y * m
    acc[0:1, :] += jnp.sum(ym, axis=0, keepdims=True)
    acc[1:2, :] += jnp.sum(ym * y, axis=0, keepdims=True)

    @pl.when(pid == pl.num_programs(0) - 1)
    def _():
        st_ref[...] = jnp.zeros_like(st_ref)
        st_ref[0:2, :] = acc[...]


def _norm_mm_body(nvalid, scale_out, y_ref, st_ref, g_ref, be_ref, w_ref,
                  dinv_ref, b1_ref, b2_ref, o_ref):
    inv_n = 1.0 / nvalid
    mu = st_ref[0:1, :] * inv_n
    var = st_ref[1:2, :] * inv_n - mu * mu
    istd = lax.rsqrt(var + 1e-5)
    xn = (y_ref[...] - mu) * istd * g_ref[...] + be_ref[...]
    r = jnp.maximum(xn, 0.0)
    h = lax.dot_general(r, w_ref[...], (((1,), (1,)), ((), ())),
                        preferred_element_type=jnp.float32)
    if scale_out:
        o_ref[...] = h * dinv_ref[...]
    else:
        o_ref[...] = h + b1_ref[...] + b2_ref[...]


def _lstm_body(C, H, z_ref, whh_ref, wo_ref, bo_ref, o_ref, h_s, c_s, hs_buf):
    pid = pl.program_id(0)

    @pl.when(pid == 0)
    def _():
        h_s[...] = jnp.zeros_like(h_s)
        c_s[...] = jnp.zeros_like(c_s)

    whh = whh_ref[...]

    def step2(i, carry):
        h, c = carry
        for u in range(4):
            t = i * 4 + u
            zt = z_ref[pl.ds(t, 1), :]
            z = zt + jnp.sum(whh * h.reshape(H, 1), axis=0, keepdims=True)
            gi = 0.5 + 0.5 * jnp.tanh(0.5 * z[:, 0:H])
            gf = 0.5 + 0.5 * jnp.tanh(0.5 * z[:, H:2 * H])
            gg = jnp.tanh(z[:, 2 * H:3 * H])
            go = 0.5 + 0.5 * jnp.tanh(0.5 * z[:, 3 * H:4 * H])
            c = gf * c + gi * gg
            h = go * jnp.tanh(c)
            hs_buf[pl.ds(t, 1), :] = h
        return (h, c)

    hf, cf = lax.fori_loop(0, C // 4, step2, (h_s[...], c_s[...]))
    h_s[...] = hf
    c_s[...] = cf
    out = lax.dot_general(hs_buf[...], wo_ref[...], (((1,), (1,)), ((), ())),
                          preferred_element_type=jnp.float32)
    o_ref[...] = out + bo_ref[...]


def kernel(x, edge_index, edge_weight, W1, b1, g1, be1, W2, b2, g2, be2,
           W_ih, W_hh, b_ih, b_hh, Wo, bo):
    N, D = x.shape
    H = W1.shape[0]
    E = edge_index.shape[1]
    P = Wo.shape[0]
    R = 1024
    NP = -(-N // R) * R
    EPU = _NCORES * _NTILES * _CHUNK * 12
    EP = -(-E // EPU) * EPU

    src = edge_index[0]
    dst = edge_index[1]
    pe = EP - E
    if pe:
        src = jnp.concatenate([src, jnp.zeros((pe,), src.dtype)])
        dst = jnp.concatenate([dst, jnp.zeros((pe,), dst.dtype)])
        ew = jnp.concatenate([edge_weight, jnp.zeros((pe,), edge_weight.dtype)])
    else:
        ew = edge_weight
    xp = jnp.pad(x, ((0, NP - N), (0, 0)))

    deg = _make_deg_dinv(NP, EP)(dst, ew)
    dinv = pl.pallas_call(
        _dinv_body,
        out_shape=jax.ShapeDtypeStruct((1, NP), jnp.float32),
    )(deg.reshape(_NCORES, NP))
    dinv_col = dinv.reshape(NP, 1)

    G = NP // R
    row_spec = pl.BlockSpec((R, H), lambda i: (i, 0))
    col_spec = pl.BlockSpec((R, 1), lambda i: (i, 0))
    w_spec = lambda shp: pl.BlockSpec(shp, lambda i: (0,) * len(shp))

    h1s = pl.pallas_call(
        _mm_scale_body,
        grid=(G,),
        in_specs=[pl.BlockSpec((R, D), lambda i: (i, 0)), w_spec((H, D)), col_spec],
        out_specs=row_spec,
        out_shape=jax.ShapeDtypeStruct((NP, H), jnp.float32),
    )(xp, W1, dinv_col)

    agg = _make_agg(NP, EP, H)

    def post(hs, p, b):
        return pl.pallas_call(
            functools.partial(_post_body, N, R),
            grid=(G,),
            in_specs=[row_spec,
                      pl.BlockSpec((2, R, H), lambda i: (0, i, 0)),
                      col_spec, w_spec((1, H))],
            out_specs=[row_spec, pl.BlockSpec((8, H), lambda i: (0, 0))],
            out_shape=[jax.ShapeDtypeStruct((NP, H), jnp.float32),
                       jax.ShapeDtypeStruct((8, H), jnp.float32)],
            scratch_shapes=[pltpu.VMEM((2, H), jnp.float32)],
        )(hs, p, dinv_col, b.reshape(1, H))

    def norm_mm(y, st, g, be, w, b1b, b2b, scale_out, KOUT):
        return pl.pallas_call(
            functools.partial(_norm_mm_body, N, scale_out),
            grid=(G,),
            in_specs=[row_spec, w_spec((8, H)), w_spec((1, H)), w_spec((1, H)),
                      w_spec((KOUT, H)), col_spec, w_spec((1, KOUT)),
                      w_spec((1, KOUT))],
            out_specs=pl.BlockSpec((R, KOUT), lambda i: (i, 0)),
            out_shape=jax.ShapeDtypeStruct((NP, KOUT), jnp.float32),
        )(y, st, g.reshape(1, H), be.reshape(1, H), w, dinv_col,
          b1b.reshape(1, KOUT), b2b.reshape(1, KOUT))

    p1 = agg(h1s, src, dst, ew)
    y1, st1 = post(h1s, p1, b1)
    zero_h = jnp.zeros((H,), jnp.float32)
    h2s = norm_mm(y1, st1, g1, be1, W2, zero_h, zero_h, True, H)
    p2 = agg(h2s, src, dst, ew)
    y2, st2 = post(h2s, p2, b2)
    zpre = norm_mm(y2, st2, g2, be2, W_ih, b_ih, b_hh, False, 4 * H)

    C = 1000 if N % 1000 == 0 else N
    GL = N // C
    out = pl.pallas_call(
        functools.partial(_lstm_body, C, H),
        grid=(GL,),
        in_specs=[pl.BlockSpec((C, 4 * H), lambda i: (i, 0)),
                  w_spec((H, 4 * H)), w_spec((P, H)), w_spec((1, P))],
        out_specs=pl.BlockSpec((C, P), lambda i: (i, 0)),
        out_shape=jax.ShapeDtypeStruct((N, P), jnp.float32),
        scratch_shapes=[pltpu.VMEM((1, H), jnp.float32),
                        pltpu.VMEM((1, H), jnp.float32),
                        pltpu.VMEM((C, H), jnp.float32)],
    )(zpre, W_hh.T, Wo, bo.reshape(1, P))
    return out
